# baseline (speedup 1.0000x reference)
import jax
import jax.numpy as jnp
from jax import lax
from jax.experimental import pallas as pl
from jax.experimental.pallas import tpu as pltpu
from jax.experimental.pallas import tpu_sc as plsc

N = 10000
E = 320000
P = 3
D = 128
T = 2000
ALPHA = 0.01

NC, NS = 2, 16
NW = NC * NS
EW = E // NW
OC = 2000
NOC = EW // OC
CH = 128
KCAP = EW + CH
TACC = 2048
SPB = TACC // NW
W = 256

_f32 = jnp.float32
_i32 = jnp.int32


def _s_body(f_ref, a_ref, o_ref):
    o_ref[...] = jnp.sum(f_ref[...] * a_ref[...], axis=1, keepdims=True)


_s_call = pl.pallas_call(
    _s_body,
    out_shape=jax.ShapeDtypeStruct((N, 1), _f32),
)

_sc_params = pltpu.CompilerParams(needs_layout_passes=False)
_sc_mesh = plsc.VectorSubcoreMesh(core_axis_name="c", subcore_axis_name="s",
                                  num_cores=NC, num_subcores=NS)


def _ka_body(s_hbm, edgef, dst_hbm, tgt_hbm,
             ke0d, ke1d, ke2d, ksld, kwd, cntd, rmd,
             slot_v, s_v, tgt_v, dstc, edgec,
             ke0, ke1, ke2, kslot, kw, cnt_v, rmbuf):
    c = lax.axis_index("c")
    sid = lax.axis_index("s")
    w = sid * NC + c
    iota = lax.iota(_i32, 16)

    pltpu.sync_copy(tgt_hbm, tgt_v)
    pltpu.sync_copy(s_hbm, s_v)

    neg1 = jnp.full((16,), -1, _i32)

    def _init(g, carry):
        slot_v[pl.ds(g * 16, 16)] = neg1
        return carry

    lax.fori_loop(0, N // 16, _init, 0)

    def _mark(g, carry):
        idx = tgt_v[pl.ds(g * 16, 16)]
        plsc.store_scatter(slot_v, [idx], g * 16 + iota)
        return carry

    lax.fori_loop(0, T // 16, _mark, 0)

    def _oc_body(oc, cnt):
        base = w * EW + oc * OC
        pltpu.sync_copy(dst_hbm.at[pl.ds(base, OC)], dstc)
        pltpu.sync_copy(edgef.at[pl.ds(base * 3, OC * 3)], edgec)

        def _grp(g, cnt2):
            dv = dstc[pl.ds(g * 16, 16)]
            sl = plsc.load_gather(slot_v, [dv])
            keep = sl >= 0
            eb = (g * 16 + iota) * 3
            e0 = plsc.load_gather(edgec, [eb])
            e1 = plsc.load_gather(edgec, [eb + 1])
            e2 = plsc.load_gather(edgec, [eb + 2])
            s0 = plsc.load_gather(s_v, [e0])
            s1 = plsc.load_gather(s_v, [e1])
            s2 = plsc.load_gather(s_v, [e2])
            a = (s0 + s1 + s2) * (1.0 / 3.0)
            a = jnp.where(a >= 0.0, a, ALPHA * a)
            wv = jnp.exp(a)
            plsc.store_compressed(ke0.at[pl.ds(cnt2, 16)], e0, mask=keep)
            plsc.store_compressed(ke1.at[pl.ds(cnt2, 16)], e1, mask=keep)
            plsc.store_compressed(ke2.at[pl.ds(cnt2, 16)], e2, mask=keep)
            plsc.store_compressed(kslot.at[pl.ds(cnt2, 16)], sl, mask=keep)
            plsc.store_compressed(kw.at[pl.ds(cnt2, 16)], wv, mask=keep)
            pc = plsc.all_reduce_population_count(keep)
            return cnt2 + jnp.max(pc)

        return lax.fori_loop(0, OC // 16, _grp, cnt)

    cnt = lax.fori_loop(0, NOC, _oc_body, jnp.int32(0))

    pltpu.sync_copy(ke0.at[pl.ds(0, EW)], ke0d.at[pl.ds(w * EW, EW)])
    pltpu.sync_copy(ke1.at[pl.ds(0, EW)], ke1d.at[pl.ds(w * EW, EW)])
    pltpu.sync_copy(ke2.at[pl.ds(0, EW)], ke2d.at[pl.ds(w * EW, EW)])
    pltpu.sync_copy(kslot.at[pl.ds(0, EW)], ksld.at[pl.ds(w * EW, EW)])
    pltpu.sync_copy(kw.at[pl.ds(0, EW)], kwd.at[pl.ds(w * EW, EW)])
    cnt_v[pl.ds(0, 16)] = jnp.full((16,), 1, _i32) * cnt
    pltpu.sync_copy(cnt_v, cntd.at[w])

    for k in range(SPB // 16):
        g = w * (SPB // 16) + k

        @pl.when(g < T // 16)
        def _():
            tg = tgt_v[pl.ds(g * 16, 16)]
            rmbuf[pl.ds(k * 16, 16)] = plsc.load_gather(slot_v, [tg])

        @pl.when(g >= T // 16)
        def _():
            rmbuf[pl.ds(k * 16, 16)] = jnp.zeros((16,), _i32)

    pltpu.sync_copy(rmbuf, rmd.at[pl.ds(w * SPB, SPB)])


_ka_call = pl.kernel(
    _ka_body,
    out_type=(
        jax.ShapeDtypeStruct((E + CH,), _i32),
        jax.ShapeDtypeStruct((E + CH,), _i32),
        jax.ShapeDtypeStruct((E + CH,), _i32),
        jax.ShapeDtypeStruct((E + CH,), _i32),
        jax.ShapeDtypeStruct((E + CH,), _f32),
        jax.ShapeDtypeStruct((NW, 16), _i32),
        jax.ShapeDtypeStruct((TACC,), _i32),
    ),
    mesh=_sc_mesh,
    compiler_params=_sc_params,
    scratch_types=[
        pltpu.VMEM((N,), _i32),
        pltpu.VMEM((N,), _f32),
        pltpu.VMEM((T,), _i32),
        pltpu.VMEM((OC,), _i32),
        pltpu.VMEM((OC * 3,), _i32),
        pltpu.VMEM((KCAP,), _i32),
        pltpu.VMEM((KCAP,), _i32),
        pltpu.VMEM((KCAP,), _i32),
        pltpu.VMEM((KCAP,), _i32),
        pltpu.VMEM((KCAP,), _f32),
        pltpu.VMEM((16,), _i32),
        pltpu.VMEM((SPB,), _i32),
    ],
)


def _kb_body(feat, ke0d, ke1d, ke2d, ksld, kwd, cntd,
             accd,
             lsl, lw, le0, le1, le2,
             me0, me1, me2, msl, mw, cnt_v,
             e0i, e1i, e2i, wch, slch,
             buf0, buf1, buf2, acc,
             sem0, sem1, sem2):
    c = lax.axis_index("c")
    sid = lax.axis_index("s")
    w = sid * NC + c
    base = w * SPB
    iota = lax.iota(_i32, 16)
    z16f = jnp.zeros((16,), _f32)
    z16i = jnp.zeros((16,), _i32)

    def _zacc(i, carry):
        for q in range(W // 16):
            acc[i, pl.ds(q * 16, 16)] = z16f
        return carry

    lax.fori_loop(0, SPB, _zacc, 0)

    def _seg_body(seg, carry):
        pltpu.sync_copy(cntd.at[seg], cnt_v)
        cseg = jnp.max(cnt_v[pl.ds(0, 16)])
        nchunk = (cseg + OC - 1) // OC

        def _oc_body(oc, carry2):
            sbase = seg * EW + oc * OC
            cpa = pltpu.async_copy(ksld.at[pl.ds(sbase, OC)], lsl, sem0)
            cpb = pltpu.async_copy(kwd.at[pl.ds(sbase, OC)], lw, sem0)
            cpc = pltpu.async_copy(ke0d.at[pl.ds(sbase, OC)], le0, sem0)
            cpd = pltpu.async_copy(ke1d.at[pl.ds(sbase, OC)], le1, sem0)
            cpe = pltpu.async_copy(ke2d.at[pl.ds(sbase, OC)], le2, sem0)
            cpa.wait()
            cpb.wait()
            cpc.wait()
            cpd.wait()
            cpe.wait()
            lim = cseg - oc * OC
            ngrp = (jnp.minimum(lim, OC) + 15) // 16

            def _grp(g, m):
                sl = lsl[pl.ds(g * 16, 16)]
                mine = jnp.logical_and(
                    jnp.logical_and(sl >= base, sl < base + SPB),
                    g * 16 + iota < lim)
                plsc.store_compressed(me0.at[pl.ds(m, 16)],
                                      le0[pl.ds(g * 16, 16)], mask=mine)
                plsc.store_compressed(me1.at[pl.ds(m, 16)],
                                      le1[pl.ds(g * 16, 16)], mask=mine)
                plsc.store_compressed(me2.at[pl.ds(m, 16)],
                                      le2[pl.ds(g * 16, 16)], mask=mine)
                plsc.store_compressed(msl.at[pl.ds(m, 16)], sl - base,
                                      mask=mine)
                plsc.store_compressed(mw.at[pl.ds(m, 16)],
                                      lw[pl.ds(g * 16, 16)], mask=mine)
                pc = plsc.all_reduce_population_count(mine)
                return m + jnp.max(pc)

            mcnt = lax.fori_loop(0, ngrp, _grp, jnp.int32(0))

            for k in range(CH // 16):
                me0[pl.ds(mcnt + k * 16, 16)] = z16i
                me1[pl.ds(mcnt + k * 16, 16)] = z16i
                me2[pl.ds(mcnt + k * 16, 16)] = z16i
                msl[pl.ds(mcnt + k * 16, 16)] = z16i
                mw[pl.ds(mcnt + k * 16, 16)] = z16f

            nch = (mcnt + CH - 1) // CH

            def _ch_body(ch, carry3):
                off = ch * CH
                for q in range(CH // 16):
                    e0i[pl.ds(q * 16, 16)] = me0[pl.ds(off + q * 16, 16)]
                    e1i[pl.ds(q * 16, 16)] = me1[pl.ds(off + q * 16, 16)]
                    e2i[pl.ds(q * 16, 16)] = me2[pl.ds(off + q * 16, 16)]
                    wch[pl.ds(q * 16, 16)] = mw[pl.ds(off + q * 16, 16)]
                    slch[pl.ds(q * 16, 16)] = msl[pl.ds(off + q * 16, 16)]
                cp0 = pltpu.async_copy(feat.at[e0i], buf0, sem0)
                cp1 = pltpu.async_copy(feat.at[e1i], buf1, sem1)
                cp2 = pltpu.async_copy(feat.at[e2i], buf2, sem2)
                cp0.wait()
                cp1.wait()
                cp2.wait()

                def _row(i, carry4):
                    i16 = jnp.full((16,), 1, _i32) * i
                    wb = plsc.load_gather(wch, [i16])
                    wb3 = wb * (1.0 / 3.0)
                    r = jnp.max(plsc.load_gather(slch, [i16]))
                    for cb in range(D // 16):
                        acc[r, pl.ds(cb * 16, 16)] = (
                            acc[r, pl.ds(cb * 16, 16)]
                            + (buf0[i, pl.ds(cb * 16, 16)]
                               + buf1[i, pl.ds(cb * 16, 16)]
                               + buf2[i, pl.ds(cb * 16, 16)]) * wb3)
                    acc[r, pl.ds(D, 16)] = (
                        acc[r, pl.ds(D, 16)] + jnp.where(iota == 0, wb, 0.0))
                    return carry4

                lax.fori_loop(0, CH, _row, 0)
                return carry3

            lax.fori_loop(0, nch, _ch_body, 0)
            return carry2

        lax.fori_loop(0, nchunk, _oc_body, 0)
        return carry

    lax.fori_loop(0, NW, _seg_body, 0)

    pltpu.sync_copy(acc, accd.at[pl.ds(base, SPB)])


_kb_call = pl.kernel(
    _kb_body,
    out_type=jax.ShapeDtypeStruct((TACC, W), _f32),
    mesh=_sc_mesh,
    compiler_params=_sc_params,
    scratch_types=[
        pltpu.VMEM((OC,), _i32),
        pltpu.VMEM((OC,), _f32),
        pltpu.VMEM((OC,), _i32),
        pltpu.VMEM((OC,), _i32),
        pltpu.VMEM((OC,), _i32),
        pltpu.VMEM((OC + CH,), _i32),
        pltpu.VMEM((OC + CH,), _i32),
        pltpu.VMEM((OC + CH,), _i32),
        pltpu.VMEM((OC + CH,), _i32),
        pltpu.VMEM((OC + CH,), _f32),
        pltpu.VMEM((16,), _i32),
        pltpu.VMEM((CH,), _i32),
        pltpu.VMEM((CH,), _i32),
        pltpu.VMEM((CH,), _i32),
        pltpu.VMEM((CH,), _f32),
        pltpu.VMEM((CH,), _i32),
        pltpu.VMEM((CH, D), _f32),
        pltpu.VMEM((CH, D), _f32),
        pltpu.VMEM((CH, D), _f32),
        pltpu.VMEM((SPB, W), _f32),
        pltpu.SemaphoreType.DMA,
        pltpu.SemaphoreType.DMA,
        pltpu.SemaphoreType.DMA,
    ],
)


def _kc_body(accd, rmd, outp, rm_v, b0, ob, sem0):
    c = lax.axis_index("c")
    sid = lax.axis_index("s")
    w = sid * NC + c

    pltpu.sync_copy(rmd.at[pl.ds(w * SPB, SPB)], rm_v)
    pltpu.async_copy(accd.at[rm_v], b0, sem0).wait()

    def _row(i, carry):
        i16 = jnp.full((16,), 1, _i32) * i
        d16 = jnp.full((16,), 1, _i32) * D
        dn = plsc.load_gather(b0, [i16, d16])
        inv = 1.0 / jnp.where(dn > 0.0, dn, 1.0)
        for cb in range(D // 16):
            ob[i, pl.ds(cb * 16, 16)] = b0[i, pl.ds(cb * 16, 16)] * inv
        return carry

    lax.fori_loop(0, SPB, _row, 0)
    pltpu.sync_copy(ob, outp.at[pl.ds(w * SPB, SPB)])


_kc_call = pl.kernel(
    _kc_body,
    out_type=jax.ShapeDtypeStruct((TACC, D), _f32),
    mesh=_sc_mesh,
    compiler_params=_sc_params,
    scratch_types=[
        pltpu.VMEM((SPB,), _i32),
        pltpu.VMEM((SPB, W), _f32),
        pltpu.VMEM((SPB, D), _f32),
        pltpu.SemaphoreType.DMA,
    ],
)


@jax.jit
def kernel(features, type_mask, edge, dst, target_idx, attn):
    del type_mask
    s = _s_call(features, attn).reshape(N)
    ke0d, ke1d, ke2d, ksld, kwd, cntd, rmd = _ka_call(
        s, edge.reshape(E * P), dst, target_idx)
    accd = _kb_call(features, ke0d, ke1d, ke2d, ksld, kwd, cntd)
    outp = _kc_call(accd, rmd)
    return outp[:T]


# speedup vs baseline: 3.1474x; 3.1474x over previous
import jax
import jax.numpy as jnp
from jax import lax
from jax.experimental import pallas as pl
from jax.experimental.pallas import tpu as pltpu
from jax.experimental.pallas import tpu_sc as plsc

N = 10000
E = 320000
P = 3
D = 128
T = 2000
ALPHA = 0.01

NC, NS = 2, 16
NW = NC * NS
EW = E // NW
OC = 2000
NOC = EW // OC
CH = 64
KCAP = EW + CH
TACC = 2048
SPB = TACC // NW
W = 256

_f32 = jnp.float32
_i32 = jnp.int32


def _s_body(f_ref, a_ref, o_ref):
    o_ref[...] = jnp.sum(f_ref[...] * a_ref[...], axis=1, keepdims=True)


_s_call = pl.pallas_call(
    _s_body,
    out_shape=jax.ShapeDtypeStruct((N, 1), _f32),
)

_sc_params = pltpu.CompilerParams(needs_layout_passes=False)
_sc_mesh = plsc.VectorSubcoreMesh(core_axis_name="c", subcore_axis_name="s",
                                  num_cores=NC, num_subcores=NS)


def _ka_body(s_hbm, edgef, dst_hbm, tgt_hbm,
             ke0d, ke1d, ke2d, ksld, kwd, cntd, rmd,
             slot_v, s_v, tgt_v, dstc, edgec,
             ke0, ke1, ke2, kslot, kw, cnt_v, rmbuf):
    c = lax.axis_index("c")
    sid = lax.axis_index("s")
    w = sid * NC + c
    iota = lax.iota(_i32, 16)

    pltpu.sync_copy(tgt_hbm, tgt_v)
    pltpu.sync_copy(s_hbm, s_v)

    neg1 = jnp.full((16,), -1, _i32)

    def _init(g, carry):
        slot_v[pl.ds(g * 16, 16)] = neg1
        return carry

    lax.fori_loop(0, N // 16, _init, 0)

    def _mark(g, carry):
        idx = tgt_v[pl.ds(g * 16, 16)]
        plsc.store_scatter(slot_v, [idx], g * 16 + iota)
        return carry

    lax.fori_loop(0, T // 16, _mark, 0)

    def _oc_body(oc, cnt):
        base = w * EW + oc * OC
        pltpu.sync_copy(dst_hbm.at[pl.ds(base, OC)], dstc)
        pltpu.sync_copy(edgef.at[pl.ds(base * 3, OC * 3)], edgec)

        def _grp(g, cnt2):
            dv = dstc[pl.ds(g * 16, 16)]
            sl = plsc.load_gather(slot_v, [dv])
            keep = sl >= 0
            eb = (g * 16 + iota) * 3
            e0 = plsc.load_gather(edgec, [eb])
            e1 = plsc.load_gather(edgec, [eb + 1])
            e2 = plsc.load_gather(edgec, [eb + 2])
            s0 = plsc.load_gather(s_v, [e0])
            s1 = plsc.load_gather(s_v, [e1])
            s2 = plsc.load_gather(s_v, [e2])
            a = (s0 + s1 + s2) * (1.0 / 3.0)
            a = jnp.where(a >= 0.0, a, ALPHA * a)
            wv = jnp.exp(a)
            plsc.store_compressed(ke0.at[pl.ds(cnt2, 16)], e0, mask=keep)
            plsc.store_compressed(ke1.at[pl.ds(cnt2, 16)], e1, mask=keep)
            plsc.store_compressed(ke2.at[pl.ds(cnt2, 16)], e2, mask=keep)
            plsc.store_compressed(kslot.at[pl.ds(cnt2, 16)], sl, mask=keep)
            plsc.store_compressed(kw.at[pl.ds(cnt2, 16)], wv, mask=keep)
            pc = plsc.all_reduce_population_count(keep)
            return cnt2 + jnp.max(pc)

        return lax.fori_loop(0, OC // 16, _grp, cnt)

    cnt = lax.fori_loop(0, NOC, _oc_body, jnp.int32(0))

    pltpu.sync_copy(ke0.at[pl.ds(0, EW)], ke0d.at[pl.ds(w * EW, EW)])
    pltpu.sync_copy(ke1.at[pl.ds(0, EW)], ke1d.at[pl.ds(w * EW, EW)])
    pltpu.sync_copy(ke2.at[pl.ds(0, EW)], ke2d.at[pl.ds(w * EW, EW)])
    pltpu.sync_copy(kslot.at[pl.ds(0, EW)], ksld.at[pl.ds(w * EW, EW)])
    pltpu.sync_copy(kw.at[pl.ds(0, EW)], kwd.at[pl.ds(w * EW, EW)])
    cnt_v[pl.ds(0, 16)] = jnp.full((16,), 1, _i32) * cnt
    pltpu.sync_copy(cnt_v, cntd.at[w])

    for k in range(SPB // 16):
        g = w * (SPB // 16) + k

        @pl.when(g < T // 16)
        def _():
            tg = tgt_v[pl.ds(g * 16, 16)]
            rmbuf[pl.ds(k * 16, 16)] = plsc.load_gather(slot_v, [tg])

        @pl.when(g >= T // 16)
        def _():
            rmbuf[pl.ds(k * 16, 16)] = jnp.zeros((16,), _i32)

    pltpu.sync_copy(rmbuf, rmd.at[pl.ds(w * SPB, SPB)])


_ka_call = pl.kernel(
    _ka_body,
    out_type=(
        jax.ShapeDtypeStruct((E + CH,), _i32),
        jax.ShapeDtypeStruct((E + CH,), _i32),
        jax.ShapeDtypeStruct((E + CH,), _i32),
        jax.ShapeDtypeStruct((E + CH,), _i32),
        jax.ShapeDtypeStruct((E + CH,), _f32),
        jax.ShapeDtypeStruct((NW, 16), _i32),
        jax.ShapeDtypeStruct((TACC,), _i32),
    ),
    mesh=_sc_mesh,
    compiler_params=_sc_params,
    scratch_types=[
        pltpu.VMEM((N,), _i32),
        pltpu.VMEM((N,), _f32),
        pltpu.VMEM((T,), _i32),
        pltpu.VMEM((OC,), _i32),
        pltpu.VMEM((OC * 3,), _i32),
        pltpu.VMEM((KCAP,), _i32),
        pltpu.VMEM((KCAP,), _i32),
        pltpu.VMEM((KCAP,), _i32),
        pltpu.VMEM((KCAP,), _i32),
        pltpu.VMEM((KCAP,), _f32),
        pltpu.VMEM((16,), _i32),
        pltpu.VMEM((SPB,), _i32),
    ],
)


def _kb_body(feat, ke0d, ke1d, ke2d, ksld, kwd, cntd,
             accd,
             lsl, lw, le0, le1, le2,
             me0, me1, me2, msl, mw, cnt_v,
             e0i, e1i, e2i, wch, slch,
             buf0, buf1, buf2, acc,
             sem0, sem1, sem2):
    c = lax.axis_index("c")
    sid = lax.axis_index("s")
    w = sid * NC + c
    base = w * SPB
    iota = lax.iota(_i32, 16)
    z16f = jnp.zeros((16,), _f32)
    z16i = jnp.zeros((16,), _i32)

    def _zacc(i, carry):
        for q in range(W // 16):
            acc[i, pl.ds(q * 16, 16)] = z16f
        return carry

    lax.fori_loop(0, SPB, _zacc, 0)

    def _seg_body(seg, carry):
        pltpu.sync_copy(cntd.at[seg], cnt_v)
        cseg = jnp.max(cnt_v[pl.ds(0, 16)])
        nchunk = (cseg + OC - 1) // OC

        def _oc_body(oc, carry2):
            sbase = seg * EW + oc * OC
            cpa = pltpu.async_copy(ksld.at[pl.ds(sbase, OC)], lsl, sem0)
            cpb = pltpu.async_copy(kwd.at[pl.ds(sbase, OC)], lw, sem0)
            cpc = pltpu.async_copy(ke0d.at[pl.ds(sbase, OC)], le0, sem0)
            cpd = pltpu.async_copy(ke1d.at[pl.ds(sbase, OC)], le1, sem0)
            cpe = pltpu.async_copy(ke2d.at[pl.ds(sbase, OC)], le2, sem0)
            cpa.wait()
            cpb.wait()
            cpc.wait()
            cpd.wait()
            cpe.wait()
            lim = cseg - oc * OC
            ngrp = (jnp.minimum(lim, OC) + 15) // 16

            def _grp(g, m):
                sl = lsl[pl.ds(g * 16, 16)]
                mine = jnp.logical_and(
                    jnp.logical_and(sl >= base, sl < base + SPB),
                    g * 16 + iota < lim)
                plsc.store_compressed(me0.at[pl.ds(m, 16)],
                                      le0[pl.ds(g * 16, 16)], mask=mine)
                plsc.store_compressed(me1.at[pl.ds(m, 16)],
                                      le1[pl.ds(g * 16, 16)], mask=mine)
                plsc.store_compressed(me2.at[pl.ds(m, 16)],
                                      le2[pl.ds(g * 16, 16)], mask=mine)
                plsc.store_compressed(msl.at[pl.ds(m, 16)], sl - base,
                                      mask=mine)
                plsc.store_compressed(mw.at[pl.ds(m, 16)],
                                      lw[pl.ds(g * 16, 16)], mask=mine)
                pc = plsc.all_reduce_population_count(mine)
                return m + jnp.max(pc)

            mcnt = lax.fori_loop(0, ngrp, _grp, jnp.int32(0))

            for k in range(CH // 16):
                me0[pl.ds(mcnt + k * 16, 16)] = z16i
                me1[pl.ds(mcnt + k * 16, 16)] = z16i
                me2[pl.ds(mcnt + k * 16, 16)] = z16i
                msl[pl.ds(mcnt + k * 16, 16)] = z16i
                mw[pl.ds(mcnt + k * 16, 16)] = z16f

            nch = (mcnt + CH - 1) // CH

            def _ch_body(ch, carry3):
                off = ch * CH
                for q in range(CH // 16):
                    e0i[pl.ds(q * 16, 16)] = me0[pl.ds(off + q * 16, 16)]
                    e1i[pl.ds(q * 16, 16)] = me1[pl.ds(off + q * 16, 16)]
                    e2i[pl.ds(q * 16, 16)] = me2[pl.ds(off + q * 16, 16)]
                    wch[pl.ds(q * 16, 16)] = mw[pl.ds(off + q * 16, 16)]
                    slch[pl.ds(q * 16, 16)] = msl[pl.ds(off + q * 16, 16)]
                cp0 = pltpu.async_copy(feat.at[e0i], buf0, sem0)
                cp1 = pltpu.async_copy(feat.at[e1i], buf1, sem1)
                cp2 = pltpu.async_copy(feat.at[e2i], buf2, sem2)
                cp0.wait()
                cp1.wait()
                cp2.wait()

                def _row(i, carry4):
                    i16 = jnp.full((16,), 1, _i32) * i
                    wb = plsc.load_gather(wch, [i16])
                    wb3 = wb * (1.0 / 3.0)
                    r = jnp.max(plsc.load_gather(slch, [i16]))
                    for cb in range(D // 16):
                        acc[r, pl.ds(cb * 16, 16)] = (
                            acc[r, pl.ds(cb * 16, 16)]
                            + (buf0[i, pl.ds(cb * 16, 16)]
                               + buf1[i, pl.ds(cb * 16, 16)]
                               + buf2[i, pl.ds(cb * 16, 16)]) * wb3)
                    acc[r, pl.ds(D, 16)] = (
                        acc[r, pl.ds(D, 16)] + jnp.where(iota == 0, wb, 0.0))
                    return carry4

                lax.fori_loop(0, CH, _row, 0)
                return carry3

            lax.fori_loop(0, nch, _ch_body, 0)
            return carry2

        lax.fori_loop(0, nchunk, _oc_body, 0)
        return carry

    lax.fori_loop(0, NW, _seg_body, 0)

    pltpu.sync_copy(acc, accd.at[pl.ds(base, SPB)])


_kb_call = pl.kernel(
    _kb_body,
    out_type=jax.ShapeDtypeStruct((TACC, W), _f32),
    mesh=_sc_mesh,
    compiler_params=_sc_params,
    scratch_types=[
        pltpu.VMEM((OC,), _i32),
        pltpu.VMEM((OC,), _f32),
        pltpu.VMEM((OC,), _i32),
        pltpu.VMEM((OC,), _i32),
        pltpu.VMEM((OC,), _i32),
        pltpu.VMEM((OC + CH,), _i32),
        pltpu.VMEM((OC + CH,), _i32),
        pltpu.VMEM((OC + CH,), _i32),
        pltpu.VMEM((OC + CH,), _i32),
        pltpu.VMEM((OC + CH,), _f32),
        pltpu.VMEM((16,), _i32),
        pltpu.VMEM((CH,), _i32),
        pltpu.VMEM((CH,), _i32),
        pltpu.VMEM((CH,), _i32),
        pltpu.VMEM((CH,), _f32),
        pltpu.VMEM((CH,), _i32),
        pltpu.VMEM((CH, D), _f32),
        pltpu.VMEM((CH, D), _f32),
        pltpu.VMEM((CH, D), _f32),
        pltpu.VMEM((SPB, W), _f32),
        pltpu.SemaphoreType.DMA,
        pltpu.SemaphoreType.DMA,
        pltpu.SemaphoreType.DMA,
    ],
)


def _kc_body(accd, rmd, outp, rm_v, b0, ob, sem0):
    c = lax.axis_index("c")
    sid = lax.axis_index("s")
    w = sid * NC + c

    pltpu.sync_copy(rmd.at[pl.ds(w * SPB, SPB)], rm_v)
    pltpu.async_copy(accd.at[rm_v], b0, sem0).wait()

    def _row(i, carry):
        i16 = jnp.full((16,), 1, _i32) * i
        d16 = jnp.full((16,), 1, _i32) * D
        dn = plsc.load_gather(b0, [i16, d16])
        inv = 1.0 / jnp.where(dn > 0.0, dn, 1.0)
        for cb in range(D // 16):
            ob[i, pl.ds(cb * 16, 16)] = b0[i, pl.ds(cb * 16, 16)] * inv
        return carry

    lax.fori_loop(0, SPB, _row, 0)
    pltpu.sync_copy(ob, outp.at[pl.ds(w * SPB, SPB)])


_kc_call = pl.kernel(
    _kc_body,
    out_type=jax.ShapeDtypeStruct((TACC, D), _f32),
    mesh=_sc_mesh,
    compiler_params=_sc_params,
    scratch_types=[
        pltpu.VMEM((SPB,), _i32),
        pltpu.VMEM((SPB, W), _f32),
        pltpu.VMEM((SPB, D), _f32),
        pltpu.SemaphoreType.DMA,
    ],
)


@jax.jit
def kernel(features, type_mask, edge, dst, target_idx, attn):
    del type_mask
    s = _s_call(features, attn).reshape(N)
    ke0d, ke1d, ke2d, ksld, kwd, cntd, rmd = _ka_call(
        s, edge.reshape(E * P), dst, target_idx)
    accd = _kb_call(features, ke0d, ke1d, ke2d, ksld, kwd, cntd)
    outp = _kc_call(accd, rmd)
    return outp[:T]


# speedup vs baseline: 3.1476x; 1.0001x over previous
import jax
import jax.numpy as jnp
from jax import lax
from jax.experimental import pallas as pl
from jax.experimental.pallas import tpu as pltpu
from jax.experimental.pallas import tpu_sc as plsc

N = 10000
E = 320000
P = 3
D = 128
T = 2000
ALPHA = 0.01

NC, NS = 2, 16
NW = NC * NS
EW = E // NW
OC = 2000
NOC = EW // OC
CH = 64
KCAP = EW + CH
TACC = 2048
SPB = TACC // NW
W = 256

_f32 = jnp.float32
_i32 = jnp.int32


def _s_body(f_ref, a_ref, o_ref):
    o_ref[...] = jnp.sum(f_ref[...] * a_ref[...], axis=1, keepdims=True)


_s_call = pl.pallas_call(
    _s_body,
    out_shape=jax.ShapeDtypeStruct((N, 1), _f32),
)

_sc_params = pltpu.CompilerParams(needs_layout_passes=False)
_sc_mesh = plsc.VectorSubcoreMesh(core_axis_name="c", subcore_axis_name="s",
                                  num_cores=NC, num_subcores=NS)


def _ka_body(s_hbm, edgef, dst_hbm, tgt_hbm,
             ke0d, ke1d, ke2d, ksld, kwd, cntd, rmd,
             slot_v, s_v, tgt_v, dstc, edgec,
             ke0, ke1, ke2, kslot, kw, cnt_v, rmbuf):
    c = lax.axis_index("c")
    sid = lax.axis_index("s")
    w = sid * NC + c
    iota = lax.iota(_i32, 16)

    pltpu.sync_copy(tgt_hbm, tgt_v)
    pltpu.sync_copy(s_hbm, s_v)

    neg1 = jnp.full((16,), -1, _i32)

    def _init(g, carry):
        slot_v[pl.ds(g * 16, 16)] = neg1
        return carry

    lax.fori_loop(0, N // 16, _init, 0)

    def _mark(g, carry):
        idx = tgt_v[pl.ds(g * 16, 16)]
        plsc.store_scatter(slot_v, [idx], g * 16 + iota)
        return carry

    lax.fori_loop(0, T // 16, _mark, 0)

    def _oc_body(oc, cnt):
        base = w * EW + oc * OC
        pltpu.sync_copy(dst_hbm.at[pl.ds(base, OC)], dstc)
        pltpu.sync_copy(edgef.at[pl.ds(base * 3, OC * 3)], edgec)

        def _grp(g, cnt2):
            dv = dstc[pl.ds(g * 16, 16)]
            sl = plsc.load_gather(slot_v, [dv])
            keep = sl >= 0
            eb = (g * 16 + iota) * 3
            e0 = plsc.load_gather(edgec, [eb])
            e1 = plsc.load_gather(edgec, [eb + 1])
            e2 = plsc.load_gather(edgec, [eb + 2])
            s0 = plsc.load_gather(s_v, [e0])
            s1 = plsc.load_gather(s_v, [e1])
            s2 = plsc.load_gather(s_v, [e2])
            a = (s0 + s1 + s2) * (1.0 / 3.0)
            a = jnp.where(a >= 0.0, a, ALPHA * a)
            wv = jnp.exp(a)
            plsc.store_compressed(ke0.at[pl.ds(cnt2, 16)], e0, mask=keep)
            plsc.store_compressed(ke1.at[pl.ds(cnt2, 16)], e1, mask=keep)
            plsc.store_compressed(ke2.at[pl.ds(cnt2, 16)], e2, mask=keep)
            plsc.store_compressed(kslot.at[pl.ds(cnt2, 16)], sl, mask=keep)
            plsc.store_compressed(kw.at[pl.ds(cnt2, 16)], wv, mask=keep)
            pc = plsc.all_reduce_population_count(keep)
            return cnt2 + jnp.max(pc)

        return lax.fori_loop(0, OC // 16, _grp, cnt)

    cnt = lax.fori_loop(0, NOC, _oc_body, jnp.int32(0))

    pltpu.sync_copy(ke0.at[pl.ds(0, EW)], ke0d.at[pl.ds(w * EW, EW)])
    pltpu.sync_copy(ke1.at[pl.ds(0, EW)], ke1d.at[pl.ds(w * EW, EW)])
    pltpu.sync_copy(ke2.at[pl.ds(0, EW)], ke2d.at[pl.ds(w * EW, EW)])
    pltpu.sync_copy(kslot.at[pl.ds(0, EW)], ksld.at[pl.ds(w * EW, EW)])
    pltpu.sync_copy(kw.at[pl.ds(0, EW)], kwd.at[pl.ds(w * EW, EW)])
    cnt_v[pl.ds(0, 16)] = jnp.full((16,), 1, _i32) * cnt
    pltpu.sync_copy(cnt_v, cntd.at[w])

    for k in range(SPB // 16):
        g = w * (SPB // 16) + k

        @pl.when(g < T // 16)
        def _():
            tg = tgt_v[pl.ds(g * 16, 16)]
            rmbuf[pl.ds(k * 16, 16)] = plsc.load_gather(slot_v, [tg])

        @pl.when(g >= T // 16)
        def _():
            rmbuf[pl.ds(k * 16, 16)] = jnp.zeros((16,), _i32)

    pltpu.sync_copy(rmbuf, rmd.at[pl.ds(w * SPB, SPB)])


_ka_call = pl.kernel(
    _ka_body,
    out_type=(
        jax.ShapeDtypeStruct((E + CH,), _i32),
        jax.ShapeDtypeStruct((E + CH,), _i32),
        jax.ShapeDtypeStruct((E + CH,), _i32),
        jax.ShapeDtypeStruct((E + CH,), _i32),
        jax.ShapeDtypeStruct((E + CH,), _f32),
        jax.ShapeDtypeStruct((NW, 16), _i32),
        jax.ShapeDtypeStruct((TACC,), _i32),
    ),
    mesh=_sc_mesh,
    compiler_params=_sc_params,
    scratch_types=[
        pltpu.VMEM((N,), _i32),
        pltpu.VMEM((N,), _f32),
        pltpu.VMEM((T,), _i32),
        pltpu.VMEM((OC,), _i32),
        pltpu.VMEM((OC * 3,), _i32),
        pltpu.VMEM((KCAP,), _i32),
        pltpu.VMEM((KCAP,), _i32),
        pltpu.VMEM((KCAP,), _i32),
        pltpu.VMEM((KCAP,), _i32),
        pltpu.VMEM((KCAP,), _f32),
        pltpu.VMEM((16,), _i32),
        pltpu.VMEM((SPB,), _i32),
    ],
)


def _kb_body(feat, ke0d, ke1d, ke2d, ksld, kwd, cntd,
             accd,
             lsl, lw, le0, le1, le2,
             me0, me1, me2, msl, mw, cnt_v,
             e0i, e1i, e2i, wch, slch,
             buf0, buf1, buf2, acc,
             sem0, sem1, sem2):
    c = lax.axis_index("c")
    sid = lax.axis_index("s")
    w = sid * NC + c
    base = w * SPB
    iota = lax.iota(_i32, 16)
    z16f = jnp.zeros((16,), _f32)
    z16i = jnp.zeros((16,), _i32)

    def _zacc(i, carry):
        for q in range(W // 16):
            acc[i, pl.ds(q * 16, 16)] = z16f
        return carry

    lax.fori_loop(0, SPB, _zacc, 0)

    def _seg_body(seg, carry):
        pltpu.sync_copy(cntd.at[seg], cnt_v)
        cseg = jnp.max(cnt_v[pl.ds(0, 16)])
        nchunk = (cseg + OC - 1) // OC

        def _oc_body(oc, carry2):
            sbase = seg * EW + oc * OC
            cpa = pltpu.async_copy(ksld.at[pl.ds(sbase, OC)], lsl, sem0)
            cpb = pltpu.async_copy(kwd.at[pl.ds(sbase, OC)], lw, sem0)
            cpc = pltpu.async_copy(ke0d.at[pl.ds(sbase, OC)], le0, sem0)
            cpd = pltpu.async_copy(ke1d.at[pl.ds(sbase, OC)], le1, sem0)
            cpe = pltpu.async_copy(ke2d.at[pl.ds(sbase, OC)], le2, sem0)
            cpa.wait()
            cpb.wait()
            cpc.wait()
            cpd.wait()
            cpe.wait()
            lim = cseg - oc * OC
            ngrp = (jnp.minimum(lim, OC) + 15) // 16

            def _grp(g, m):
                sl = lsl[pl.ds(g * 16, 16)]
                mine = jnp.logical_and(
                    jnp.logical_and(sl >= base, sl < base + SPB),
                    g * 16 + iota < lim)
                plsc.store_compressed(me0.at[pl.ds(m, 16)],
                                      le0[pl.ds(g * 16, 16)], mask=mine)
                plsc.store_compressed(me1.at[pl.ds(m, 16)],
                                      le1[pl.ds(g * 16, 16)], mask=mine)
                plsc.store_compressed(me2.at[pl.ds(m, 16)],
                                      le2[pl.ds(g * 16, 16)], mask=mine)
                plsc.store_compressed(msl.at[pl.ds(m, 16)], sl - base,
                                      mask=mine)
                plsc.store_compressed(mw.at[pl.ds(m, 16)],
                                      lw[pl.ds(g * 16, 16)], mask=mine)
                pc = plsc.all_reduce_population_count(mine)
                return m + jnp.max(pc)

            with jax.named_scope("b_scan"):
                mcnt = lax.fori_loop(0, ngrp, _grp, jnp.int32(0))

            for k in range(CH // 16):
                me0[pl.ds(mcnt + k * 16, 16)] = z16i
                me1[pl.ds(mcnt + k * 16, 16)] = z16i
                me2[pl.ds(mcnt + k * 16, 16)] = z16i
                msl[pl.ds(mcnt + k * 16, 16)] = z16i
                mw[pl.ds(mcnt + k * 16, 16)] = z16f

            nch = (mcnt + CH - 1) // CH

            def _ch_body(ch, carry3):
                off = ch * CH
                for q in range(CH // 16):
                    e0i[pl.ds(q * 16, 16)] = me0[pl.ds(off + q * 16, 16)]
                    e1i[pl.ds(q * 16, 16)] = me1[pl.ds(off + q * 16, 16)]
                    e2i[pl.ds(q * 16, 16)] = me2[pl.ds(off + q * 16, 16)]
                    wch[pl.ds(q * 16, 16)] = mw[pl.ds(off + q * 16, 16)]
                    slch[pl.ds(q * 16, 16)] = msl[pl.ds(off + q * 16, 16)]
                with jax.named_scope("b_gath"):
                    cp0 = pltpu.async_copy(feat.at[e0i], buf0, sem0)
                    cp1 = pltpu.async_copy(feat.at[e1i], buf1, sem1)
                    cp2 = pltpu.async_copy(feat.at[e2i], buf2, sem2)
                    cp0.wait()
                    cp1.wait()
                    cp2.wait()

                def _row(i, carry4):
                    i16 = jnp.full((16,), 1, _i32) * i
                    wb = plsc.load_gather(wch, [i16])
                    wb3 = wb * (1.0 / 3.0)
                    r = jnp.max(plsc.load_gather(slch, [i16]))
                    for cb in range(D // 16):
                        acc[r, pl.ds(cb * 16, 16)] = (
                            acc[r, pl.ds(cb * 16, 16)]
                            + (buf0[i, pl.ds(cb * 16, 16)]
                               + buf1[i, pl.ds(cb * 16, 16)]
                               + buf2[i, pl.ds(cb * 16, 16)]) * wb3)
                    acc[r, pl.ds(D, 16)] = (
                        acc[r, pl.ds(D, 16)] + jnp.where(iota == 0, wb, 0.0))
                    return carry4

                with jax.named_scope("b_rmw"):
                    lax.fori_loop(0, CH, _row, 0)
                return carry3

            lax.fori_loop(0, nch, _ch_body, 0)
            return carry2

        lax.fori_loop(0, nchunk, _oc_body, 0)
        return carry

    lax.fori_loop(0, NW, _seg_body, 0)

    pltpu.sync_copy(acc, accd.at[pl.ds(base, SPB)])


_kb_call = pl.kernel(
    _kb_body,
    out_type=jax.ShapeDtypeStruct((TACC, W), _f32),
    mesh=_sc_mesh,
    compiler_params=_sc_params,
    scratch_types=[
        pltpu.VMEM((OC,), _i32),
        pltpu.VMEM((OC,), _f32),
        pltpu.VMEM((OC,), _i32),
        pltpu.VMEM((OC,), _i32),
        pltpu.VMEM((OC,), _i32),
        pltpu.VMEM((OC + CH,), _i32),
        pltpu.VMEM((OC + CH,), _i32),
        pltpu.VMEM((OC + CH,), _i32),
        pltpu.VMEM((OC + CH,), _i32),
        pltpu.VMEM((OC + CH,), _f32),
        pltpu.VMEM((16,), _i32),
        pltpu.VMEM((CH,), _i32),
        pltpu.VMEM((CH,), _i32),
        pltpu.VMEM((CH,), _i32),
        pltpu.VMEM((CH,), _f32),
        pltpu.VMEM((CH,), _i32),
        pltpu.VMEM((CH, D), _f32),
        pltpu.VMEM((CH, D), _f32),
        pltpu.VMEM((CH, D), _f32),
        pltpu.VMEM((SPB, W), _f32),
        pltpu.SemaphoreType.DMA,
        pltpu.SemaphoreType.DMA,
        pltpu.SemaphoreType.DMA,
    ],
)


def _kc_body(accd, rmd, outp, rm_v, b0, ob, sem0):
    c = lax.axis_index("c")
    sid = lax.axis_index("s")
    w = sid * NC + c

    pltpu.sync_copy(rmd.at[pl.ds(w * SPB, SPB)], rm_v)
    pltpu.async_copy(accd.at[rm_v], b0, sem0).wait()

    def _row(i, carry):
        i16 = jnp.full((16,), 1, _i32) * i
        d16 = jnp.full((16,), 1, _i32) * D
        dn = plsc.load_gather(b0, [i16, d16])
        inv = 1.0 / jnp.where(dn > 0.0, dn, 1.0)
        for cb in range(D // 16):
            ob[i, pl.ds(cb * 16, 16)] = b0[i, pl.ds(cb * 16, 16)] * inv
        return carry

    lax.fori_loop(0, SPB, _row, 0)
    pltpu.sync_copy(ob, outp.at[pl.ds(w * SPB, SPB)])


_kc_call = pl.kernel(
    _kc_body,
    out_type=jax.ShapeDtypeStruct((TACC, D), _f32),
    mesh=_sc_mesh,
    compiler_params=_sc_params,
    scratch_types=[
        pltpu.VMEM((SPB,), _i32),
        pltpu.VMEM((SPB, W), _f32),
        pltpu.VMEM((SPB, D), _f32),
        pltpu.SemaphoreType.DMA,
    ],
)


@jax.jit
def kernel(features, type_mask, edge, dst, target_idx, attn):
    del type_mask
    s = _s_call(features, attn).reshape(N)
    ke0d, ke1d, ke2d, ksld, kwd, cntd, rmd = _ka_call(
        s, edge.reshape(E * P), dst, target_idx)
    accd = _kb_call(features, ke0d, ke1d, ke2d, ksld, kwd, cntd)
    outp = _kc_call(accd, rmd)
    return outp[:T]


# speedup vs baseline: 13.3714x; 4.2481x over previous
import jax
import jax.numpy as jnp
from jax import lax
from jax.experimental import pallas as pl
from jax.experimental.pallas import tpu as pltpu
from jax.experimental.pallas import tpu_sc as plsc

N = 10000
E = 320000
P = 3
D = 128
T = 2000
ALPHA = 0.01

NC, NS = 2, 16
NW = NC * NS
EW = E // NW
OC = 2000
NOC = EW // OC
CH = 32
AW = 256
KCAP = EW + CH
TACC = 2048
SPB = TACC // NW
W = 256

_f32 = jnp.float32
_i32 = jnp.int32


def _s_body(f_ref, a_ref, o_ref):
    o_ref[...] = jnp.sum(f_ref[...] * a_ref[...], axis=1, keepdims=True)


_s_call = pl.pallas_call(
    _s_body,
    out_shape=jax.ShapeDtypeStruct((N, 1), _f32),
)

_sc_params = pltpu.CompilerParams(needs_layout_passes=False)
_sc_mesh = plsc.VectorSubcoreMesh(core_axis_name="c", subcore_axis_name="s",
                                  num_cores=NC, num_subcores=NS)


def _ka_body(s_hbm, edgef, dst_hbm, tgt_hbm,
             ke0d, ke1d, ke2d, ksld, kwd, cntd, rmd,
             slot_v, s_v, tgt_v, dstc, edgec,
             ke0, ke1, ke2, kslot, kw, cnt_v, rmbuf):
    c = lax.axis_index("c")
    sid = lax.axis_index("s")
    w = sid * NC + c
    iota = lax.iota(_i32, 16)

    pltpu.sync_copy(tgt_hbm, tgt_v)
    pltpu.sync_copy(s_hbm, s_v)

    neg1 = jnp.full((16,), -1, _i32)

    def _init(g, carry):
        slot_v[pl.ds(g * 16, 16)] = neg1
        return carry

    lax.fori_loop(0, N // 16, _init, 0)

    def _mark(g, carry):
        idx = tgt_v[pl.ds(g * 16, 16)]
        plsc.store_scatter(slot_v, [idx], g * 16 + iota)
        return carry

    lax.fori_loop(0, T // 16, _mark, 0)

    def _oc_body(oc, cnt):
        base = w * EW + oc * OC
        pltpu.sync_copy(dst_hbm.at[pl.ds(base, OC)], dstc)
        pltpu.sync_copy(edgef.at[pl.ds(base * 3, OC * 3)], edgec)

        def _grp(g, cnt2):
            dv = dstc[pl.ds(g * 16, 16)]
            sl = plsc.load_gather(slot_v, [dv])
            keep = sl >= 0
            eb = (g * 16 + iota) * 3
            e0 = plsc.load_gather(edgec, [eb])
            e1 = plsc.load_gather(edgec, [eb + 1])
            e2 = plsc.load_gather(edgec, [eb + 2])
            s0 = plsc.load_gather(s_v, [e0])
            s1 = plsc.load_gather(s_v, [e1])
            s2 = plsc.load_gather(s_v, [e2])
            a = (s0 + s1 + s2) * (1.0 / 3.0)
            a = jnp.where(a >= 0.0, a, ALPHA * a)
            wv = jnp.exp(a)
            plsc.store_compressed(ke0.at[pl.ds(cnt2, 16)], e0, mask=keep)
            plsc.store_compressed(ke1.at[pl.ds(cnt2, 16)], e1, mask=keep)
            plsc.store_compressed(ke2.at[pl.ds(cnt2, 16)], e2, mask=keep)
            plsc.store_compressed(kslot.at[pl.ds(cnt2, 16)], sl, mask=keep)
            plsc.store_compressed(kw.at[pl.ds(cnt2, 16)], wv, mask=keep)
            pc = plsc.all_reduce_population_count(keep)
            return cnt2 + jnp.max(pc)

        return lax.fori_loop(0, OC // 16, _grp, cnt)

    cnt = lax.fori_loop(0, NOC, _oc_body, jnp.int32(0))

    pltpu.sync_copy(ke0.at[pl.ds(0, EW)], ke0d.at[pl.ds(w * EW, EW)])
    pltpu.sync_copy(ke1.at[pl.ds(0, EW)], ke1d.at[pl.ds(w * EW, EW)])
    pltpu.sync_copy(ke2.at[pl.ds(0, EW)], ke2d.at[pl.ds(w * EW, EW)])
    pltpu.sync_copy(kslot.at[pl.ds(0, EW)], ksld.at[pl.ds(w * EW, EW)])
    pltpu.sync_copy(kw.at[pl.ds(0, EW)], kwd.at[pl.ds(w * EW, EW)])
    cnt_v[pl.ds(0, 16)] = jnp.full((16,), 1, _i32) * cnt
    pltpu.sync_copy(cnt_v, cntd.at[w])

    for k in range(SPB // 16):
        g = w * (SPB // 16) + k

        @pl.when(g < T // 16)
        def _():
            tg = tgt_v[pl.ds(g * 16, 16)]
            rmbuf[pl.ds(k * 16, 16)] = plsc.load_gather(slot_v, [tg])

        @pl.when(g >= T // 16)
        def _():
            rmbuf[pl.ds(k * 16, 16)] = jnp.zeros((16,), _i32)

    pltpu.sync_copy(rmbuf, rmd.at[pl.ds(w * SPB, SPB)])


_ka_call = pl.kernel(
    _ka_body,
    out_type=(
        jax.ShapeDtypeStruct((E + CH,), _i32),
        jax.ShapeDtypeStruct((E + CH,), _i32),
        jax.ShapeDtypeStruct((E + CH,), _i32),
        jax.ShapeDtypeStruct((E + CH,), _i32),
        jax.ShapeDtypeStruct((E + CH,), _f32),
        jax.ShapeDtypeStruct((NW, 16), _i32),
        jax.ShapeDtypeStruct((TACC,), _i32),
    ),
    mesh=_sc_mesh,
    compiler_params=_sc_params,
    scratch_types=[
        pltpu.VMEM((N,), _i32),
        pltpu.VMEM((N,), _f32),
        pltpu.VMEM((T,), _i32),
        pltpu.VMEM((OC,), _i32),
        pltpu.VMEM((OC * 3,), _i32),
        pltpu.VMEM((KCAP,), _i32),
        pltpu.VMEM((KCAP,), _i32),
        pltpu.VMEM((KCAP,), _i32),
        pltpu.VMEM((KCAP,), _i32),
        pltpu.VMEM((KCAP,), _f32),
        pltpu.VMEM((16,), _i32),
        pltpu.VMEM((SPB,), _i32),
    ],
)


def _kb_body(feat, ke0d, ke1d, ke2d, ksld, kwd, cntd,
             accd,
             lsl, lw, le0, le1, le2,
             me0, me1, me2, msl, mw, cnt_v,
             e0i, e1i, e2i, wch, slch,
             buf0, buf1, buf2, acc, feat_sh,
             sem0, sem1, sem2):
    c = lax.axis_index("c")
    sid = lax.axis_index("s")
    w = sid * NC + c
    base = w * SPB
    iota = lax.iota(_i32, 16)
    z16f = jnp.zeros((16,), _f32)
    z16i = jnp.zeros((16,), _i32)

    @pl.when(sid < 10)
    def _():
        pltpu.sync_copy(feat.at[pl.ds(sid * 1000, 1000)],
                        feat_sh.at[pl.ds(sid * 1000, 1000)])

    def _zacc(i, carry):
        for q in range(AW // 16):
            acc[i, pl.ds(q * 16, 16)] = z16f
        return carry

    lax.fori_loop(0, SPB, _zacc, 0)
    plsc.subcore_barrier()

    def _seg_body(seg, carry):
        pltpu.sync_copy(cntd.at[seg], cnt_v)
        cseg = jnp.max(cnt_v[pl.ds(0, 16)])
        nchunk = (cseg + OC - 1) // OC

        def _oc_body(oc, carry2):
            sbase = seg * EW + oc * OC
            cpa = pltpu.async_copy(ksld.at[pl.ds(sbase, OC)], lsl, sem0)
            cpb = pltpu.async_copy(kwd.at[pl.ds(sbase, OC)], lw, sem0)
            cpc = pltpu.async_copy(ke0d.at[pl.ds(sbase, OC)], le0, sem0)
            cpd = pltpu.async_copy(ke1d.at[pl.ds(sbase, OC)], le1, sem0)
            cpe = pltpu.async_copy(ke2d.at[pl.ds(sbase, OC)], le2, sem0)
            cpa.wait()
            cpb.wait()
            cpc.wait()
            cpd.wait()
            cpe.wait()
            lim = cseg - oc * OC
            ngrp = (jnp.minimum(lim, OC) + 15) // 16

            def _grp(g, m):
                sl = lsl[pl.ds(g * 16, 16)]
                mine = jnp.logical_and(
                    jnp.logical_and(sl >= base, sl < base + SPB),
                    g * 16 + iota < lim)
                plsc.store_compressed(me0.at[pl.ds(m, 16)],
                                      le0[pl.ds(g * 16, 16)], mask=mine)
                plsc.store_compressed(me1.at[pl.ds(m, 16)],
                                      le1[pl.ds(g * 16, 16)], mask=mine)
                plsc.store_compressed(me2.at[pl.ds(m, 16)],
                                      le2[pl.ds(g * 16, 16)], mask=mine)
                plsc.store_compressed(msl.at[pl.ds(m, 16)], sl - base,
                                      mask=mine)
                plsc.store_compressed(mw.at[pl.ds(m, 16)],
                                      lw[pl.ds(g * 16, 16)], mask=mine)
                pc = plsc.all_reduce_population_count(mine)
                return m + jnp.max(pc)

            with jax.named_scope("b_scan"):
                mcnt = lax.fori_loop(0, ngrp, _grp, jnp.int32(0))

            for k in range(CH // 16):
                me0[pl.ds(mcnt + k * 16, 16)] = z16i
                me1[pl.ds(mcnt + k * 16, 16)] = z16i
                me2[pl.ds(mcnt + k * 16, 16)] = z16i
                msl[pl.ds(mcnt + k * 16, 16)] = z16i
                mw[pl.ds(mcnt + k * 16, 16)] = z16f

            nch = (mcnt + CH - 1) // CH

            def _ch_body(ch, carry3):
                off = ch * CH
                for q in range(CH // 16):
                    e0i[pl.ds(q * 16, 16)] = me0[pl.ds(off + q * 16, 16)]
                    e1i[pl.ds(q * 16, 16)] = me1[pl.ds(off + q * 16, 16)]
                    e2i[pl.ds(q * 16, 16)] = me2[pl.ds(off + q * 16, 16)]
                    wch[pl.ds(q * 16, 16)] = mw[pl.ds(off + q * 16, 16)]
                    slch[pl.ds(q * 16, 16)] = msl[pl.ds(off + q * 16, 16)]
                with jax.named_scope("b_gath"):
                    cp0 = pltpu.async_copy(feat_sh.at[e0i], buf0, sem0)
                    cp1 = pltpu.async_copy(feat_sh.at[e1i], buf1, sem1)
                    cp2 = pltpu.async_copy(feat_sh.at[e2i], buf2, sem2)
                    cp0.wait()
                    cp1.wait()
                    cp2.wait()

                def _row(i, carry4):
                    i16 = jnp.full((16,), 1, _i32) * i
                    wb = plsc.load_gather(wch, [i16])
                    wb3 = wb * (1.0 / 3.0)
                    r = jnp.max(plsc.load_gather(slch, [i16]))
                    for cb in range(D // 16):
                        acc[r, pl.ds(cb * 16, 16)] = (
                            acc[r, pl.ds(cb * 16, 16)]
                            + (buf0[i, pl.ds(cb * 16, 16)]
                               + buf1[i, pl.ds(cb * 16, 16)]
                               + buf2[i, pl.ds(cb * 16, 16)]) * wb3)
                    acc[r, pl.ds(D, 16)] = (
                        acc[r, pl.ds(D, 16)] + jnp.where(iota == 0, wb, 0.0))
                    return carry4

                with jax.named_scope("b_rmw"):
                    lax.fori_loop(0, CH, _row, 0)
                return carry3

            lax.fori_loop(0, nch, _ch_body, 0)
            return carry2

        lax.fori_loop(0, nchunk, _oc_body, 0)
        return carry

    lax.fori_loop(0, NW, _seg_body, 0)

    pltpu.sync_copy(acc, accd.at[pl.ds(base, SPB), pl.ds(0, AW)])


_kb_call = pl.kernel(
    _kb_body,
    out_type=jax.ShapeDtypeStruct((TACC, W), _f32),
    mesh=_sc_mesh,
    compiler_params=_sc_params,
    scratch_types=[
        pltpu.VMEM((OC,), _i32),
        pltpu.VMEM((OC,), _f32),
        pltpu.VMEM((OC,), _i32),
        pltpu.VMEM((OC,), _i32),
        pltpu.VMEM((OC,), _i32),
        pltpu.VMEM((OC + CH,), _i32),
        pltpu.VMEM((OC + CH,), _i32),
        pltpu.VMEM((OC + CH,), _i32),
        pltpu.VMEM((OC + CH,), _i32),
        pltpu.VMEM((OC + CH,), _f32),
        pltpu.VMEM((16,), _i32),
        pltpu.VMEM((CH,), _i32),
        pltpu.VMEM((CH,), _i32),
        pltpu.VMEM((CH,), _i32),
        pltpu.VMEM((CH,), _f32),
        pltpu.VMEM((CH,), _i32),
        pltpu.VMEM((CH, D), _f32),
        pltpu.VMEM((CH, D), _f32),
        pltpu.VMEM((CH, D), _f32),
        pltpu.VMEM((SPB, AW), _f32),
        pltpu.VMEM_SHARED((N, D), _f32),
        pltpu.SemaphoreType.DMA,
        pltpu.SemaphoreType.DMA,
        pltpu.SemaphoreType.DMA,
    ],
)


def _kc_body(accd, rmd, outp, rm_v, b0, ob, sem0):
    c = lax.axis_index("c")
    sid = lax.axis_index("s")
    w = sid * NC + c

    pltpu.sync_copy(rmd.at[pl.ds(w * SPB, SPB)], rm_v)
    pltpu.async_copy(accd.at[rm_v], b0, sem0).wait()

    def _row(i, carry):
        i16 = jnp.full((16,), 1, _i32) * i
        d16 = jnp.full((16,), 1, _i32) * D
        dn = plsc.load_gather(b0, [i16, d16])
        inv = 1.0 / jnp.where(dn > 0.0, dn, 1.0)
        for cb in range(D // 16):
            ob[i, pl.ds(cb * 16, 16)] = b0[i, pl.ds(cb * 16, 16)] * inv
        return carry

    lax.fori_loop(0, SPB, _row, 0)
    pltpu.sync_copy(ob, outp.at[pl.ds(w * SPB, SPB)])


_kc_call = pl.kernel(
    _kc_body,
    out_type=jax.ShapeDtypeStruct((TACC, D), _f32),
    mesh=_sc_mesh,
    compiler_params=_sc_params,
    scratch_types=[
        pltpu.VMEM((SPB,), _i32),
        pltpu.VMEM((SPB, W), _f32),
        pltpu.VMEM((SPB, D), _f32),
        pltpu.SemaphoreType.DMA,
    ],
)


@jax.jit
def kernel(features, type_mask, edge, dst, target_idx, attn):
    del type_mask
    s = _s_call(features, attn).reshape(N)
    ke0d, ke1d, ke2d, ksld, kwd, cntd, rmd = _ka_call(
        s, edge.reshape(E * P), dst, target_idx)
    accd = _kb_call(features, ke0d, ke1d, ke2d, ksld, kwd, cntd)
    outp = _kc_call(accd, rmd)
    return outp[:T]
